# baseline (speedup 1.0000x reference)
import functools

import jax
import jax.numpy as jnp
from jax import lax
from jax.experimental import pallas as pl
from jax.experimental.pallas import tpu as pltpu
from jax.experimental.pallas import tpu_sc as plsc

T = 2048
D = 1024
HID = 2048
E = 8
K = 2
B = 256
G = (T * K) // B + E
GB = G * B

_F32 = jnp.float32
_I32 = jnp.int32


def _router_body(x_ref, wr_ref, mi_ref, mf_ref, be_ref, aux_ref, xb_ref):
    xf = x_ref[...]
    xb_ref[...] = xf.astype(jnp.bfloat16)
    wr = wr_ref[...]
    logits = lax.dot_general(
        xf, wr, (((1,), (1,)), ((), ())),
        preferred_element_type=_F32)

    m = jnp.max(logits, axis=-1, keepdims=True)
    ex = jnp.exp(logits - m)
    probs = ex / jnp.sum(ex, axis=-1, keepdims=True)

    lane = lax.broadcasted_iota(_I32, (T, E), 1)
    is1 = logits == jnp.max(logits, axis=-1, keepdims=True)
    i1 = jnp.min(jnp.where(is1, lane, E), axis=-1, keepdims=True)
    oh1 = lane == i1
    l2 = jnp.where(oh1, -jnp.inf, logits)
    is2 = l2 == jnp.max(l2, axis=-1, keepdims=True)
    i2 = jnp.min(jnp.where(is2, lane, E), axis=-1, keepdims=True)
    oh2 = lane == i2

    p1 = jnp.sum(jnp.where(oh1, probs, 0.0), axis=-1, keepdims=True)
    p2 = jnp.sum(jnp.where(oh2, probs, 0.0), axis=-1, keepdims=True)
    s = jnp.clip(p1 + p2, 1e-9, None)
    w1 = p1 / s
    w2 = p2 / s

    density = jnp.mean(probs, axis=0, keepdims=True)
    proxy = jnp.mean((probs > (1.0 / E)).astype(_F32), axis=0, keepdims=True)
    aux_ref[...] = jnp.sum(density * proxy, keepdims=True) * float(E * E)

    oh1f = oh1.astype(_F32)
    oh2f = oh2.astype(_F32)
    CH = 256
    rid = lax.broadcasted_iota(_I32, (CH, CH), 0)
    cid = lax.broadcasted_iota(_I32, (CH, CH), 1)
    tril = (rid >= cid).astype(_F32)

    def chunked_cumsum(oh, carry):
        outs = []
        for c in range(T // CH):
            blk = oh[c * CH:(c + 1) * CH, :]
            cs = lax.dot_general(
                tril, blk, (((1,), (0,)), ((), ())),
                preferred_element_type=_F32, precision=lax.Precision.HIGHEST)
            outs.append(cs + carry)
            carry = carry + cs[CH - 1:CH, :]
        return jnp.concatenate(outs, axis=0), carry

    cs1, c1tot = chunked_cumsum(oh1f, jnp.zeros((1, E), _F32))
    cs2, ctot = chunked_cumsum(oh2f, c1tot)

    c_row = ctot
    cpad_row = jnp.ceil(c_row * (1.0 / B)) * B
    er = lax.broadcasted_iota(_I32, (E, E), 0)
    ec = lax.broadcasted_iota(_I32, (E, E), 1)
    mlt = (er < ec).astype(_F32)
    o_row = lax.dot_general(
        cpad_row, mlt, (((1,), (0,)), ((), ())),
        preferred_element_type=_F32)

    pos0 = jnp.sum(oh1f * (cs1 + o_row), axis=-1, keepdims=True) - 1.0
    pos1 = jnp.sum(oh2f * (cs2 + o_row), axis=-1, keepdims=True) - 1.0

    mi_ref[...] = jnp.concatenate(
        [pos0.astype(_I32), pos1.astype(_I32),
         jnp.zeros((T, 6), _I32)], axis=1)
    mf_ref[...] = jnp.concatenate(
        [w1, w2, jnp.zeros((T, 6), _F32)], axis=1)

    ones_col = jnp.ones((T, 1), _F32)
    c_col = lax.dot_general(
        oh1f + oh2f, ones_col, (((0,), (0,)), ((), ())),
        preferred_element_type=_F32)
    cpad_col = jnp.ceil(c_col * (1.0 / B)) * B
    m2 = (ec < er).astype(_F32)
    o_col = lax.dot_general(
        m2, cpad_col, (((1,), (0,)), ((), ())),
        preferred_element_type=_F32)
    bstart = o_col * (1.0 / B)
    giota = lax.broadcasted_iota(_I32, (E, 32), 1).astype(_F32)
    cmp = (bstart <= giota).astype(_I32)
    base = jnp.sum(cmp, axis=0, keepdims=True) - 1
    nblk = jnp.sum(cpad_row * (1.0 / B), axis=-1, keepdims=True).astype(_I32)
    gsel = lax.broadcasted_iota(_I32, (1, 32), 1)
    be_ref[...] = jnp.where(gsel == G, nblk, base)


def _router_call(flat, wr):
    return pl.pallas_call(
        _router_body,
        out_shape=[
            jax.ShapeDtypeStruct((T, E), _I32),
            jax.ShapeDtypeStruct((T, E), _F32),
            jax.ShapeDtypeStruct((1, 32), _I32),
            jax.ShapeDtypeStruct((1, 1), _F32),
            jax.ShapeDtypeStruct((T, D), jnp.bfloat16),
        ],
    )(flat, wr)


@functools.cache
def _sc_mesh():
    return plsc.VectorSubcoreMesh(core_axis_name="c", subcore_axis_name="s")


def _bin_body(pos_hbm, w_hbm, idx_hbm, ws_hbm, pos_v, w_v, idx_v, ws_v):
    cid = lax.axis_index("c")
    sid = lax.axis_index("s")
    wid = sid * 2 + cid

    @pl.when(wid == 0)
    def _():
        pltpu.sync_copy(pos_hbm, pos_v)
        pltpu.sync_copy(w_hbm, w_v)
        zi = jnp.zeros((16,), _I32)
        zf = jnp.zeros((16,), _F32)

        def zero_step(i, _):
            idx_v[pl.ds(i * 16, 16)] = zi
            ws_v[pl.ds(i * 16, 16)] = zf
            return _

        lax.fori_loop(0, GB // 16, zero_step, 0)
        lane = lax.iota(_I32, 16)

        def scat_step(i, _):
            p = pos_v[pl.ds(i * 16, 16)]
            w = w_v[pl.ds(i * 16, 16)]
            tok = lax.rem(i * 16, T) + lane
            plsc.store_scatter(idx_v, [p], tok)
            plsc.store_scatter(ws_v, [p], w)
            return _

        lax.fori_loop(0, (T * K) // 16, scat_step, 0)
        pltpu.sync_copy(idx_v, idx_hbm)
        pltpu.sync_copy(ws_v, ws_hbm)


def _bin_call(pos01, w01):
    f = functools.partial(
        pl.kernel,
        out_type=[
            jax.ShapeDtypeStruct((GB,), _I32),
            jax.ShapeDtypeStruct((GB,), _F32),
        ],
        mesh=_sc_mesh(),
        compiler_params=pltpu.CompilerParams(needs_layout_passes=False),
        scratch_types=[
            pltpu.VMEM((T * K,), _I32),
            pltpu.VMEM((T * K,), _F32),
            pltpu.VMEM((GB,), _I32),
            pltpu.VMEM((GB,), _F32),
        ],
    )(_bin_body)
    return f(pos01, w01)


def _ffn_body(be_ref, idx_ref, ws_ref, x_ref, w1_ref, w3_ref, w2_ref, out_ref,
              xs_ref, yacc_ref):
    g = pl.program_id(0)
    hb = pl.program_id(1)
    nblk = be_ref[G]

    @pl.when(g < nblk)
    def _():
        w1b = w1_ref[0].astype(jnp.bfloat16)
        w3b = w3_ref[0].astype(jnp.bfloat16)
        w2b = w2_ref[0].astype(jnp.bfloat16)

        @pl.when(hb == 0)
        def _():
            ti = lax.broadcasted_iota(_I32, (T, B), 0)
            oh = jnp.where(ti == idx_ref[0], 1.0, 0.0).astype(jnp.bfloat16)
            xs_ref[...] = lax.dot_general(
                oh, x_ref[...], (((0,), (0,)), ((), ())),
                preferred_element_type=_F32).astype(jnp.bfloat16)

        xs = xs_ref[...]
        a = lax.dot_general(xs, w1b, (((1,), (1,)), ((), ())),
                            preferred_element_type=_F32)
        c = lax.dot_general(xs, w3b, (((1,), (1,)), ((), ())),
                            preferred_element_type=_F32)
        h = ((a * lax.logistic(a)) * c).astype(jnp.bfloat16)
        yp = lax.dot_general(h, w2b, (((1,), (1,)), ((), ())),
                             preferred_element_type=_F32)

        @pl.when(hb == 0)
        def _():
            yacc_ref[...] = yp

        @pl.when(hb == 1)
        def _():
            ti = lax.broadcasted_iota(_I32, (T, B), 0)
            ohw = jnp.where(ti == idx_ref[0], ws_ref[0], 0.0).astype(
                jnp.bfloat16)
            contrib = lax.dot_general(
                ohw, (yacc_ref[...] + yp).astype(jnp.bfloat16),
                (((1,), (0,)), ((), ())),
                preferred_element_type=_F32)

            @pl.when(g == 0)
            def _():
                out_ref[...] = contrib

            @pl.when(g != 0)
            def _():
                out_ref[...] += contrib


def _ffn_call(be, idx3, ws3, xb, w1b, w3b, w2b):
    grid_spec = pltpu.PrefetchScalarGridSpec(
        num_scalar_prefetch=1,
        grid=(G, 2),
        in_specs=[
            pl.BlockSpec((1, 1, B), lambda g, hb, be: (g, 0, 0)),
            pl.BlockSpec((1, 1, B), lambda g, hb, be: (g, 0, 0)),
            pl.BlockSpec((T, D), lambda g, hb, be: (0, 0)),
            pl.BlockSpec((1, HID // 2, D), lambda g, hb, be: (be[g], hb, 0)),
            pl.BlockSpec((1, HID // 2, D), lambda g, hb, be: (be[g], hb, 0)),
            pl.BlockSpec((1, D, HID // 2), lambda g, hb, be: (be[g], 0, hb)),
        ],
        out_specs=pl.BlockSpec((T, D), lambda g, hb, be: (0, 0)),
        scratch_shapes=[
            pltpu.VMEM((B, D), jnp.bfloat16),
            pltpu.VMEM((B, D), _F32),
        ],
    )
    return pl.pallas_call(
        _ffn_body,
        grid_spec=grid_spec,
        compiler_params=pltpu.CompilerParams(
            vmem_limit_bytes=63 * 1024 * 1024),
        out_shape=jax.ShapeDtypeStruct((T, D), _F32),
    )(be, idx3, ws3, xb, w1b, w3b, w2b)


def kernel(x, Wr, W1, W2, W3):
    shape = x.shape
    flat = x.reshape(T, D)

    mi, mf, be, aux, xb = _router_call(flat, Wr)
    pos01 = jnp.concatenate([mi[:, 0], mi[:, 1]], axis=0)
    w01 = jnp.concatenate([mf[:, 0], mf[:, 1]], axis=0)

    idx, ws = _bin_call(pos01, w01)

    out = _ffn_call(
        be[0, :G + 1],
        idx.reshape(G, 1, B),
        ws.reshape(G, 1, B),
        xb,
        W1,
        W3,
        W2,
    )
    return out.reshape(shape), aux.reshape(())


# speedup vs baseline: 1.0301x; 1.0301x over previous
import functools

import jax
import jax.numpy as jnp
from jax import lax
from jax.experimental import pallas as pl
from jax.experimental.pallas import tpu as pltpu
from jax.experimental.pallas import tpu_sc as plsc

T = 2048
D = 1024
HID = 2048
E = 8
K = 2
B = 256
G = (T * K) // B + E
GB = G * B

_F32 = jnp.float32
_I32 = jnp.int32


def _router_body(x_ref, wr_ref, mi_ref, mf_ref, be_ref, aux_ref):
    xf = x_ref[...]
    wr = wr_ref[...]
    logits = lax.dot_general(
        xf, wr, (((1,), (1,)), ((), ())),
        preferred_element_type=_F32)

    m = jnp.max(logits, axis=-1, keepdims=True)
    ex = jnp.exp(logits - m)
    probs = ex / jnp.sum(ex, axis=-1, keepdims=True)

    lane = lax.broadcasted_iota(_I32, (T, E), 1)
    is1 = logits == jnp.max(logits, axis=-1, keepdims=True)
    i1 = jnp.min(jnp.where(is1, lane, E), axis=-1, keepdims=True)
    oh1 = lane == i1
    l2 = jnp.where(oh1, -jnp.inf, logits)
    is2 = l2 == jnp.max(l2, axis=-1, keepdims=True)
    i2 = jnp.min(jnp.where(is2, lane, E), axis=-1, keepdims=True)
    oh2 = lane == i2

    p1 = jnp.sum(jnp.where(oh1, probs, 0.0), axis=-1, keepdims=True)
    p2 = jnp.sum(jnp.where(oh2, probs, 0.0), axis=-1, keepdims=True)
    s = jnp.clip(p1 + p2, 1e-9, None)
    w1 = p1 / s
    w2 = p2 / s

    density = jnp.mean(probs, axis=0, keepdims=True)
    proxy = jnp.mean((probs > (1.0 / E)).astype(_F32), axis=0, keepdims=True)
    aux_ref[...] = jnp.sum(density * proxy, keepdims=True) * float(E * E)

    oh1f = oh1.astype(_F32)
    oh2f = oh2.astype(_F32)
    CH = 256
    rid = lax.broadcasted_iota(_I32, (CH, CH), 0)
    cid = lax.broadcasted_iota(_I32, (CH, CH), 1)
    tril = (rid >= cid).astype(_F32)

    def chunked_cumsum(oh, carry):
        outs = []
        for c in range(T // CH):
            blk = oh[c * CH:(c + 1) * CH, :]
            cs = lax.dot_general(
                tril, blk, (((1,), (0,)), ((), ())),
                preferred_element_type=_F32)
            outs.append(cs + carry)
            carry = carry + cs[CH - 1:CH, :]
        return jnp.concatenate(outs, axis=0), carry

    cs1, c1tot = chunked_cumsum(oh1f, jnp.zeros((1, E), _F32))
    cs2, ctot = chunked_cumsum(oh2f, c1tot)

    c_row = ctot
    cpad_row = jnp.ceil(c_row * (1.0 / B)) * B
    er = lax.broadcasted_iota(_I32, (E, E), 0)
    ec = lax.broadcasted_iota(_I32, (E, E), 1)
    mlt = (er < ec).astype(_F32)
    o_row = lax.dot_general(
        cpad_row, mlt, (((1,), (0,)), ((), ())),
        preferred_element_type=_F32)

    pos0 = jnp.sum(oh1f * (cs1 + o_row), axis=-1, keepdims=True) - 1.0
    pos1 = jnp.sum(oh2f * (cs2 + o_row), axis=-1, keepdims=True) - 1.0

    mi_ref[...] = jnp.concatenate(
        [pos0.astype(_I32), pos1.astype(_I32),
         jnp.zeros((T, 6), _I32)], axis=1)
    mf_ref[...] = jnp.concatenate(
        [w1, w2, jnp.zeros((T, 6), _F32)], axis=1)

    ones_col = jnp.ones((T, 1), _F32)
    c_col = lax.dot_general(
        oh1f + oh2f, ones_col, (((0,), (0,)), ((), ())),
        preferred_element_type=_F32)
    cpad_col = jnp.ceil(c_col * (1.0 / B)) * B
    m2 = (ec < er).astype(_F32)
    o_col = lax.dot_general(
        m2, cpad_col, (((1,), (0,)), ((), ())),
        preferred_element_type=_F32)
    bstart = o_col * (1.0 / B)
    giota = lax.broadcasted_iota(_I32, (E, 32), 1).astype(_F32)
    cmp = (bstart <= giota).astype(_I32)
    base = jnp.sum(cmp, axis=0, keepdims=True) - 1
    nblk = jnp.sum(cpad_row * (1.0 / B), axis=-1, keepdims=True).astype(_I32)
    gsel = lax.broadcasted_iota(_I32, (1, 32), 1)
    be_ref[...] = jnp.where(gsel == G, nblk, base)


def _router_call(flat, wr):
    return pl.pallas_call(
        _router_body,
        out_shape=[
            jax.ShapeDtypeStruct((T, E), _I32),
            jax.ShapeDtypeStruct((T, E), _F32),
            jax.ShapeDtypeStruct((1, 32), _I32),
            jax.ShapeDtypeStruct((1, 1), _F32),
        ],
    )(flat, wr)


@functools.cache
def _sc_mesh():
    return plsc.VectorSubcoreMesh(core_axis_name="c", subcore_axis_name="s")


def _bin_body(pos_hbm, w_hbm, idx_hbm, ws_hbm, pos_v, w_v, idx_v, ws_v):
    cid = lax.axis_index("c")
    sid = lax.axis_index("s")
    wid = sid * 2 + cid

    @pl.when(wid == 0)
    def _():
        pltpu.sync_copy(pos_hbm, pos_v)
        pltpu.sync_copy(w_hbm, w_v)
        zi = jnp.zeros((16,), _I32)
        zf = jnp.zeros((16,), _F32)

        def zero_step(i, _):
            idx_v[pl.ds(i * 16, 16)] = zi
            ws_v[pl.ds(i * 16, 16)] = zf
            return _

        lax.fori_loop(0, GB // 16, zero_step, 0)
        lane = lax.iota(_I32, 16)

        def scat_step(i, _):
            p = pos_v[pl.ds(i * 16, 16)]
            w = w_v[pl.ds(i * 16, 16)]
            tok = lax.rem(i * 16, T) + lane
            plsc.store_scatter(idx_v, [p], tok)
            plsc.store_scatter(ws_v, [p], w)
            return _

        lax.fori_loop(0, (T * K) // 16, scat_step, 0)
        pltpu.sync_copy(idx_v, idx_hbm)
        pltpu.sync_copy(ws_v, ws_hbm)


def _bin_call(pos01, w01):
    f = functools.partial(
        pl.kernel,
        out_type=[
            jax.ShapeDtypeStruct((GB,), _I32),
            jax.ShapeDtypeStruct((GB,), _F32),
        ],
        mesh=_sc_mesh(),
        compiler_params=pltpu.CompilerParams(needs_layout_passes=False),
        scratch_types=[
            pltpu.VMEM((T * K,), _I32),
            pltpu.VMEM((T * K,), _F32),
            pltpu.VMEM((GB,), _I32),
            pltpu.VMEM((GB,), _F32),
        ],
    )(_bin_body)
    return f(pos01, w01)


def _ffn_body(be_ref, idx_ref, idxw_ref, wsw_ref, x_ref, w1_ref, w3_ref,
              w2_ref, out_ref, xs_ref, yacc_ref, y4_ref):
    g = pl.program_id(0)
    hb = pl.program_id(1)
    nblk = be_ref[G]

    @pl.when(g < nblk)
    def _():
        @pl.when(hb == 0)
        def _():
            ti = lax.broadcasted_iota(_I32, (T, B), 0)
            oh = jnp.where(ti == idx_ref[0], 1.0, 0.0)
            xs_ref[...] = lax.dot_general(
                oh, x_ref[...], (((0,), (0,)), ((), ())),
                preferred_element_type=_F32)

        xs = xs_ref[...]
        a = lax.dot_general(xs, w1_ref[0], (((1,), (1,)), ((), ())),
                            preferred_element_type=_F32)
        c = lax.dot_general(xs, w3_ref[0], (((1,), (1,)), ((), ())),
                            preferred_element_type=_F32)
        h = (a * lax.logistic(a)) * c
        yp = lax.dot_general(h, w2_ref[0], (((1,), (1,)), ((), ())),
                             preferred_element_type=_F32)

        @pl.when(hb == 0)
        def _():
            yacc_ref[...] = yp

        @pl.when(hb == 1)
        def _():
            y4_ref[pl.ds((lax.rem(g, 4)) * B, B), :] = yacc_ref[...] + yp

            @pl.when((lax.rem(g, 4) == 3) | (g == nblk - 1))
            def _():
                ti4 = lax.broadcasted_iota(_I32, (T, 4 * B), 0)
                ohw = jnp.where(ti4 == idxw_ref[0], wsw_ref[0], 0.0)
                contrib = lax.dot_general(
                    ohw, y4_ref[...], (((1,), (0,)), ((), ())),
                    preferred_element_type=_F32)

                @pl.when(g < 4)
                def _():
                    out_ref[...] = contrib

                @pl.when(g >= 4)
                def _():
                    out_ref[...] += contrib


def _ffn_call(be, idx3, idxw, wsw, xb, w1b, w3b, w2b):
    grid_spec = pltpu.PrefetchScalarGridSpec(
        num_scalar_prefetch=1,
        grid=(G, 2),
        in_specs=[
            pl.BlockSpec((1, 1, B), lambda g, hb, be: (g, 0, 0)),
            pl.BlockSpec((1, 1, 4 * B), lambda g, hb, be: (g // 4, 0, 0)),
            pl.BlockSpec((1, 1, 4 * B), lambda g, hb, be: (g // 4, 0, 0)),
            pl.BlockSpec((T, D), lambda g, hb, be: (0, 0)),
            pl.BlockSpec((1, HID // 2, D), lambda g, hb, be: (be[g], hb, 0)),
            pl.BlockSpec((1, HID // 2, D), lambda g, hb, be: (be[g], hb, 0)),
            pl.BlockSpec((1, D, HID // 2), lambda g, hb, be: (be[g], 0, hb)),
        ],
        out_specs=pl.BlockSpec((T, D), lambda g, hb, be: (0, 0)),
        scratch_shapes=[
            pltpu.VMEM((B, D), _F32),
            pltpu.VMEM((B, D), _F32),
            pltpu.VMEM((4 * B, D), _F32),
        ],
    )
    return pl.pallas_call(
        _ffn_body,
        grid_spec=grid_spec,
        compiler_params=pltpu.CompilerParams(
            vmem_limit_bytes=63 * 1024 * 1024),
        out_shape=jax.ShapeDtypeStruct((T, D), _F32),
    )(be, idx3, idxw, wsw, xb, w1b, w3b, w2b)


def kernel(x, Wr, W1, W2, W3):
    shape = x.shape
    flat = x.reshape(T, D)

    mi, mf, be, aux = _router_call(flat, Wr)
    pos01 = jnp.concatenate([mi[:, 0], mi[:, 1]], axis=0)
    w01 = jnp.concatenate([mf[:, 0], mf[:, 1]], axis=0)

    idx, ws = _bin_call(pos01, w01)

    out = _ffn_call(
        be[0, :G + 1],
        idx.reshape(G, 1, B),
        idx.reshape(G // 4, 1, 4 * B),
        ws.reshape(G // 4, 1, 4 * B),
        flat,
        W1,
        W3,
        W2,
    )
    return out.reshape(shape), aux.reshape(())


# speedup vs baseline: 1.1235x; 1.0907x over previous
import functools

import jax
import jax.numpy as jnp
from jax import lax
from jax.experimental import pallas as pl
from jax.experimental.pallas import tpu as pltpu
from jax.experimental.pallas import tpu_sc as plsc

T = 2048
D = 1024
HID = 2048
E = 8
K = 2
B = 256
G = (T * K) // B + E
GB = G * B

_F32 = jnp.float32
_I32 = jnp.int32


def _router_body(x_ref, wr_ref, mi_ref, mf_ref, be_ref, aux_ref, xb_ref):
    xf = x_ref[...]
    xb_ref[...] = xf.astype(jnp.bfloat16)
    wr = wr_ref[...]
    logits = lax.dot_general(
        xf, wr, (((1,), (1,)), ((), ())),
        preferred_element_type=_F32)

    m = jnp.max(logits, axis=-1, keepdims=True)
    ex = jnp.exp(logits - m)
    probs = ex / jnp.sum(ex, axis=-1, keepdims=True)

    lane = lax.broadcasted_iota(_I32, (T, E), 1)
    is1 = logits == jnp.max(logits, axis=-1, keepdims=True)
    i1 = jnp.min(jnp.where(is1, lane, E), axis=-1, keepdims=True)
    oh1 = lane == i1
    l2 = jnp.where(oh1, -jnp.inf, logits)
    is2 = l2 == jnp.max(l2, axis=-1, keepdims=True)
    i2 = jnp.min(jnp.where(is2, lane, E), axis=-1, keepdims=True)
    oh2 = lane == i2

    p1 = jnp.sum(jnp.where(oh1, probs, 0.0), axis=-1, keepdims=True)
    p2 = jnp.sum(jnp.where(oh2, probs, 0.0), axis=-1, keepdims=True)
    s = jnp.clip(p1 + p2, 1e-9, None)
    w1 = p1 / s
    w2 = p2 / s

    density = jnp.mean(probs, axis=0, keepdims=True)
    proxy = jnp.mean((probs > (1.0 / E)).astype(_F32), axis=0, keepdims=True)
    aux_ref[...] = jnp.sum(density * proxy, keepdims=True) * float(E * E)

    oh1f = oh1.astype(_F32)
    oh2f = oh2.astype(_F32)
    CH = 256
    rid = lax.broadcasted_iota(_I32, (CH, CH), 0)
    cid = lax.broadcasted_iota(_I32, (CH, CH), 1)
    tril = (rid >= cid).astype(_F32)

    def chunked_cumsum(oh, carry):
        outs = []
        for c in range(T // CH):
            blk = oh[c * CH:(c + 1) * CH, :]
            cs = lax.dot_general(
                tril, blk, (((1,), (0,)), ((), ())),
                preferred_element_type=_F32)
            outs.append(cs + carry)
            carry = carry + cs[CH - 1:CH, :]
        return jnp.concatenate(outs, axis=0), carry

    cs1, c1tot = chunked_cumsum(oh1f, jnp.zeros((1, E), _F32))
    cs2, ctot = chunked_cumsum(oh2f, c1tot)

    c_row = ctot
    cpad_row = jnp.ceil(c_row * (1.0 / B)) * B
    er = lax.broadcasted_iota(_I32, (E, E), 0)
    ec = lax.broadcasted_iota(_I32, (E, E), 1)
    mlt = (er < ec).astype(_F32)
    o_row = lax.dot_general(
        cpad_row, mlt, (((1,), (0,)), ((), ())),
        preferred_element_type=_F32)

    pos0 = jnp.sum(oh1f * (cs1 + o_row), axis=-1, keepdims=True) - 1.0
    pos1 = jnp.sum(oh2f * (cs2 + o_row), axis=-1, keepdims=True) - 1.0

    mi_ref[...] = jnp.concatenate(
        [pos0.astype(_I32), pos1.astype(_I32),
         jnp.zeros((T, 6), _I32)], axis=1)
    mf_ref[...] = jnp.concatenate(
        [w1, w2, jnp.zeros((T, 6), _F32)], axis=1)

    ones_col = jnp.ones((T, 1), _F32)
    c_col = lax.dot_general(
        oh1f + oh2f, ones_col, (((0,), (0,)), ((), ())),
        preferred_element_type=_F32)
    cpad_col = jnp.ceil(c_col * (1.0 / B)) * B
    m2 = (ec < er).astype(_F32)
    o_col = lax.dot_general(
        m2, cpad_col, (((1,), (0,)), ((), ())),
        preferred_element_type=_F32)
    bstart = o_col * (1.0 / B)
    giota = lax.broadcasted_iota(_I32, (E, 32), 1).astype(_F32)
    cmp = (bstart <= giota).astype(_I32)
    base = jnp.sum(cmp, axis=0, keepdims=True) - 1
    nblk = jnp.sum(cpad_row * (1.0 / B), axis=-1, keepdims=True).astype(_I32)
    gsel = lax.broadcasted_iota(_I32, (1, 32), 1)
    be_ref[...] = jnp.where(gsel == G, nblk, base)


def _router_call(flat, wr):
    return pl.pallas_call(
        _router_body,
        out_shape=[
            jax.ShapeDtypeStruct((T, E), _I32),
            jax.ShapeDtypeStruct((T, E), _F32),
            jax.ShapeDtypeStruct((1, 32), _I32),
            jax.ShapeDtypeStruct((1, 1), _F32),
            jax.ShapeDtypeStruct((T, D), jnp.bfloat16),
        ],
    )(flat, wr)


@functools.cache
def _sc_mesh():
    return plsc.VectorSubcoreMesh(core_axis_name="c", subcore_axis_name="s")


def _bin_body(pos_hbm, w_hbm, idx_hbm, ws_hbm, pos_v, w_v, idx_v, ws_v):
    cid = lax.axis_index("c")
    sid = lax.axis_index("s")
    wid = sid * 2 + cid

    @pl.when(wid == 0)
    def _():
        pltpu.sync_copy(pos_hbm, pos_v)
        pltpu.sync_copy(w_hbm, w_v)
        zi = jnp.zeros((16,), _I32)
        zf = jnp.zeros((16,), _F32)

        def zero_step(i, _):
            idx_v[pl.ds(i * 16, 16)] = zi
            ws_v[pl.ds(i * 16, 16)] = zf
            return _

        lax.fori_loop(0, GB // 16, zero_step, 0)
        lane = lax.iota(_I32, 16)

        def scat_step(i, _):
            p = pos_v[pl.ds(i * 16, 16)]
            w = w_v[pl.ds(i * 16, 16)]
            tok = lax.rem(i * 16, T) + lane
            plsc.store_scatter(idx_v, [p], tok)
            plsc.store_scatter(ws_v, [p], w)
            return _

        lax.fori_loop(0, (T * K) // 16, scat_step, 0)
        pltpu.sync_copy(idx_v, idx_hbm)
        pltpu.sync_copy(ws_v, ws_hbm)


def _bin_call(pos01, w01):
    f = functools.partial(
        pl.kernel,
        out_type=[
            jax.ShapeDtypeStruct((GB,), _I32),
            jax.ShapeDtypeStruct((GB,), _F32),
        ],
        mesh=_sc_mesh(),
        compiler_params=pltpu.CompilerParams(needs_layout_passes=False),
        scratch_types=[
            pltpu.VMEM((T * K,), _I32),
            pltpu.VMEM((T * K,), _F32),
            pltpu.VMEM((GB,), _I32),
            pltpu.VMEM((GB,), _F32),
        ],
    )(_bin_body)
    return f(pos01, w01)


def _ffn_body(be_ref, idx_ref, idxw_ref, wsw_ref, x_ref, w1_ref, w3_ref,
              w2_ref, out_ref, xs_ref, yacc_ref, y4_ref):
    g = pl.program_id(0)
    hb = pl.program_id(1)
    nblk = be_ref[G]

    @pl.when(g < nblk)
    def _():
        @pl.when(hb == 0)
        def _():
            ti = lax.broadcasted_iota(_I32, (T, B), 0)
            oh = jnp.where(ti == idx_ref[0], 1.0, 0.0).astype(jnp.bfloat16)
            xs_ref[...] = lax.dot_general(
                oh, x_ref[...], (((0,), (0,)), ((), ())),
                preferred_element_type=_F32)

        xs = xs_ref[...]
        a = lax.dot_general(xs, w1_ref[0], (((1,), (1,)), ((), ())),
                            preferred_element_type=_F32)
        c = lax.dot_general(xs, w3_ref[0], (((1,), (1,)), ((), ())),
                            preferred_element_type=_F32)
        h = (a * lax.logistic(a)) * c
        hbe = lax.rem(hb + g, 2)

        def store_yp(yp):
            @pl.when(hb == 0)
            def _():
                yacc_ref[...] = yp

            @pl.when(hb == 1)
            def _():
                y4_ref[pl.ds((lax.rem(g, 4)) * B, B), :] = (
                    yacc_ref[...] + yp).astype(jnp.bfloat16)

        @pl.when(hbe == 0)
        def _():
            store_yp(lax.dot_general(
                h, w2_ref[0][:, :HID // 2], (((1,), (1,)), ((), ())),
                preferred_element_type=_F32))

        @pl.when(hbe == 1)
        def _():
            store_yp(lax.dot_general(
                h, w2_ref[0][:, HID // 2:], (((1,), (1,)), ((), ())),
                preferred_element_type=_F32))

            @pl.when((lax.rem(g, 4) == 3) | (g == nblk - 1))
            def _():
                ti4 = lax.broadcasted_iota(_I32, (T, 4 * B), 0)
                ohw = jnp.where(ti4 == idxw_ref[0], wsw_ref[0],
                                0.0).astype(jnp.bfloat16)
                contrib = lax.dot_general(
                    ohw, y4_ref[...], (((1,), (0,)), ((), ())),
                    preferred_element_type=_F32)

                @pl.when(g < 4)
                def _():
                    out_ref[...] = contrib

                @pl.when(g >= 4)
                def _():
                    out_ref[...] += contrib


def _ffn_call(be, idx3, idxw, wsw, xb, w1b, w3b, w2b):
    grid_spec = pltpu.PrefetchScalarGridSpec(
        num_scalar_prefetch=1,
        grid=(G, 2),
        in_specs=[
            pl.BlockSpec((1, 1, B), lambda g, hb, be: (g, 0, 0)),
            pl.BlockSpec((1, 1, 4 * B), lambda g, hb, be: (g // 4, 0, 0)),
            pl.BlockSpec((1, 1, 4 * B), lambda g, hb, be: (g // 4, 0, 0)),
            pl.BlockSpec((T, D), lambda g, hb, be: (0, 0)),
            pl.BlockSpec((1, HID // 2, D),
                         lambda g, hb, be: (be[g], hb ^ (g % 2), 0)),
            pl.BlockSpec((1, HID // 2, D),
                         lambda g, hb, be: (be[g], hb ^ (g % 2), 0)),
            pl.BlockSpec((1, D, HID), lambda g, hb, be: (be[g], 0, 0)),
        ],
        out_specs=pl.BlockSpec((T, D), lambda g, hb, be: (0, 0)),
        scratch_shapes=[
            pltpu.VMEM((B, D), _F32),
            pltpu.VMEM((B, D), _F32),
            pltpu.VMEM((4 * B, D), jnp.bfloat16),
        ],
    )
    return pl.pallas_call(
        _ffn_body,
        grid_spec=grid_spec,
        compiler_params=pltpu.CompilerParams(
            vmem_limit_bytes=62 * 1024 * 1024),
        out_shape=jax.ShapeDtypeStruct((T, D), _F32),
    )(be, idx3, idxw, wsw, xb, w1b, w3b, w2b)


def kernel(x, Wr, W1, W2, W3):
    shape = x.shape
    flat = x.reshape(T, D)

    mi, mf, be, aux, xb = _router_call(flat, Wr)
    pos01 = jnp.concatenate([mi[:, 0], mi[:, 1]], axis=0)
    w01 = jnp.concatenate([mf[:, 0], mf[:, 1]], axis=0)

    idx, ws = _bin_call(pos01, w01)

    out = _ffn_call(
        be[0, :G + 1],
        idx.reshape(G, 1, B),
        idx.reshape(G // 4, 1, 4 * B),
        ws.reshape(G // 4, 1, 4 * B),
        xb,
        W1,
        W3,
        W2,
    )
    return out.reshape(shape), aux.reshape(())
